# baseline (speedup 1.0000x reference)
import functools

import jax
import jax.numpy as jnp
import numpy as np
from jax.experimental import pallas as pl
from jax.experimental.pallas import tpu as pltpu

_T = [float(np.tan((2 * j + 1) * np.pi / 16)) for j in range(4)]

_OFFS = [(0, 1), (-1, 1), (-1, 0), (-1, -1), (0, -1), (1, -1), (1, 0), (1, 1)]


def _shl(a):
    return jnp.concatenate([a[:, 1:], jnp.zeros_like(a[:, :1])], axis=1)


def _shr(a):
    return jnp.concatenate([jnp.zeros_like(a[:, :1]), a[:, :-1]], axis=1)


def _cshift(a, dx):
    if dx == 1:
        return _shl(a)
    if dx == -1:
        return _shr(a)
    return a


def _sup(a):
    return jnp.concatenate([a[1:], jnp.zeros_like(a[:1])], axis=0)


def _sdn(a):
    return jnp.concatenate([jnp.zeros_like(a[:1]), a[:-1]], axis=0)


def _canny_block(x_ref, t_ref, b_ref, ogx_ref, ogy_ref, omag_ref, oori_ref,
                 ote_ref, *, bh, h, w):
    i = pl.program_id(1)
    n_i = pl.num_programs(1)
    third = jnp.float32(1.0 / 3.0)
    wtop = jnp.where(i > 0, third, 0.0)
    wbot = jnp.where(i < n_i - 1, third, 0.0)

    def q(a):
        return a.astype(jnp.bfloat16).astype(jnp.float32)

    m_main = (q(x_ref[0, 0]) + q(x_ref[0, 1]) + q(x_ref[0, 2])) * third
    m_top = (q(t_ref[0, 0]) + q(t_ref[0, 1]) + q(t_ref[0, 2])) * wtop
    m_bot = (q(b_ref[0, 0]) + q(b_ref[0, 1]) + q(b_ref[0, 2])) * wbot
    m = jnp.concatenate([m_top, m_main, m_bot], axis=0)

    mu, md = _sup(m), _sdn(m)
    vsm = (mu + md) * 0.5 + m
    drow = mu - md
    gx = _shl(vsm) - _shr(vsm)
    gy = (_shl(drow) + _shr(drow)) * 0.5 + drow
    mag = jnp.sqrt(gx * gx + gy * gy)

    son = jnp.where(i > 0, 1.0, 0.0)
    sbn = jnp.where(i < n_i - 1, 1.0, 0.0)
    mag = jnp.concatenate(
        [mag[:8] * son, mag[8:bh + 8], mag[bh + 8:] * sbn], axis=0)

    t = gy / gx
    at = jnp.abs(t)
    tneg = t < 0
    c1, c2, c3, c4 = (at > _T[0]), (at > _T[1]), (at > _T[2]), (at > _T[3])
    u = (jnp.where(c1, 1.0, 0.0) + jnp.where(c2, 1.0, 0.0)
         + jnp.where(c3, 1.0, 0.0) + jnp.where(c4, 1.0, 0.0))
    kf = jnp.where(tneg, -u, u)
    ori = kf * 45.0 + 180.0

    mq = q(mag)
    mq_up, mq_dn = _sup(mq), _sdn(mq)
    n_e, n_w = _shl(mq), _shr(mq)
    n_ne, n_nw = _shl(mq_dn), _shr(mq_dn)
    n_se, n_sw = _shl(mq_up), _shr(mq_up)
    sel_lo_p = jnp.where(tneg, n_nw, n_ne)
    sel_hi_p = jnp.where(tneg, n_ne, n_nw)
    n_p = jnp.where(c4, n_e,
                    jnp.where(c3, sel_hi_p,
                              jnp.where(c2, mq_dn,
                                        jnp.where(c1, sel_lo_p, n_e))))
    sel_lo_n = jnp.where(tneg, n_se, n_sw)
    sel_hi_n = jnp.where(tneg, n_sw, n_se)
    n_n = jnp.where(c4, n_w,
                    jnp.where(c3, sel_hi_n,
                              jnp.where(c2, mq_up,
                                        jnp.where(c1, sel_lo_n, n_w))))
    dp = mq - n_p
    dn = mq - n_n
    thin = jnp.where(jnp.minimum(dp, dn) > 0.0, mag, 0.0)

    te = (jnp.where(thin > 0.5, 0.5, 0.0) + jnp.where(thin > 1.0, 0.5, 0.0))
    cs = _shl(te) + te + _shr(te)
    hs = _sup(cs) + cs + _sdn(cs)
    te_c = te[8:bh + 8]
    out_thin = jnp.where(
        te_c == 1.0, 1.0,
        jnp.where(te_c == 0.5,
                  jnp.where(hs[8:bh + 8] * 1.25 > 1.0, 1.0, 0.0), 0.0))

    ogx_ref[0, 0] = gx[8:bh + 8]
    ogy_ref[0, 0] = gy[8:bh + 8]
    omag_ref[0, 0] = mag[8:bh + 8]
    oori_ref[0, 0] = ori[8:bh + 8]
    ote_ref[0, 0] = out_thin


def _run(img):
    b, ch, h, w = img.shape
    bh = 256
    n_i = h // bh
    hb = bh // 8

    out_sds = jax.ShapeDtypeStruct((b, 1, h, w), jnp.float32)
    out_spec = pl.BlockSpec((1, 1, bh, w), lambda bi, i: (bi, 0, i, 0))
    grid = (b, n_i)
    fn = functools.partial(_canny_block, bh=bh, h=h, w=w)
    outs = pl.pallas_call(
        fn,
        grid=grid,
        in_specs=[
            pl.BlockSpec((1, ch, bh, w), lambda bi, i: (bi, 0, i, 0)),
            pl.BlockSpec((1, ch, 8, w),
                         lambda bi, i: (bi, 0, jnp.maximum(hb * i - 1, 0), 0)),
            pl.BlockSpec((1, ch, 8, w),
                         lambda bi, i: (bi, 0,
                                        jnp.minimum(hb * (i + 1), h // 8 - 1),
                                        0)),
        ],
        out_specs=[out_spec] * 5,
        out_shape=[out_sds] * 5,
        compiler_params=pltpu.CompilerParams(
            dimension_semantics=("parallel", "arbitrary")),
    )(img, img, img)
    return tuple(outs)


@jax.jit
def kernel(img):
    return _run(img)


# speedup vs baseline: 1.1436x; 1.1436x over previous
import functools

import jax
import jax.numpy as jnp
import numpy as np
from jax.experimental import pallas as pl
from jax.experimental.pallas import tpu as pltpu

_T = [float(np.tan((2 * j + 1) * np.pi / 16)) for j in range(4)]

_OFFS = [(0, 1), (-1, 1), (-1, 0), (-1, -1), (0, -1), (1, -1), (1, 0), (1, 1)]


def _shl(a):
    return jnp.concatenate([a[:, 1:], jnp.zeros_like(a[:, :1])], axis=1)


def _shr(a):
    return jnp.concatenate([jnp.zeros_like(a[:, :1]), a[:, :-1]], axis=1)


def _cshift(a, dx):
    if dx == 1:
        return _shl(a)
    if dx == -1:
        return _shr(a)
    return a


def _sup(a):
    return jnp.concatenate([a[1:], jnp.zeros_like(a[:1])], axis=0)


def _sdn(a):
    return jnp.concatenate([jnp.zeros_like(a[:1]), a[:-1]], axis=0)


def _canny_block(x_ref, t_ref, b_ref, ogx_ref, ogy_ref, omag_ref, oori_ref,
                 ote_ref, *, bh, h, w):
    i = pl.program_id(1)
    n_i = pl.num_programs(1)
    third = jnp.float32(1.0 / 3.0)
    wtop = jnp.where(i > 0, third, 0.0)
    wbot = jnp.where(i < n_i - 1, third, 0.0)

    def q(a):
        return a.astype(jnp.bfloat16).astype(jnp.float32)

    m_main = (q(x_ref[0, 0]) + q(x_ref[0, 1]) + q(x_ref[0, 2])) * third
    m_top = (q(t_ref[0, 0]) + q(t_ref[0, 1]) + q(t_ref[0, 2])) * wtop
    m_bot = (q(b_ref[0, 0]) + q(b_ref[0, 1]) + q(b_ref[0, 2])) * wbot
    m = jnp.concatenate([m_top, m_main, m_bot], axis=0)

    mu, md = _sup(m), _sdn(m)
    vsm = (mu + md) * 0.5 + m
    drow = mu - md
    gx = _shl(vsm) - _shr(vsm)
    gy = (_shl(drow) + _shr(drow)) * 0.5 + drow
    mag = jnp.sqrt(gx * gx + gy * gy)

    son = jnp.where(i > 0, 1.0, 0.0)
    sbn = jnp.where(i < n_i - 1, 1.0, 0.0)
    mag = jnp.concatenate(
        [mag[:8] * son, mag[8:bh + 8], mag[bh + 8:] * sbn], axis=0)

    t = gy / gx
    at = jnp.abs(t)
    tneg = t < 0
    c1, c2, c3, c4 = (at > _T[0]), (at > _T[1]), (at > _T[2]), (at > _T[3])
    u = (jnp.where(c1, 1.0, 0.0) + jnp.where(c2, 1.0, 0.0)
         + jnp.where(c3, 1.0, 0.0) + jnp.where(c4, 1.0, 0.0))
    kf = jnp.where(tneg, -u, u)
    ori = kf * 45.0 + 180.0

    mq = q(mag)
    mq_up, mq_dn = _sup(mq), _sdn(mq)
    n_e, n_w = _shl(mq), _shr(mq)
    n_ne, n_nw = _shl(mq_dn), _shr(mq_dn)
    n_se, n_sw = _shl(mq_up), _shr(mq_up)
    mx_ew = jnp.maximum(n_e, n_w)
    mx_ns = jnp.maximum(mq_dn, mq_up)
    mx_d1 = jnp.maximum(n_ne, n_sw)
    mx_d2 = jnp.maximum(n_nw, n_se)
    mx_lo = jnp.where(tneg, mx_d2, mx_d1)
    mx_hi = jnp.where(tneg, mx_d1, mx_d2)
    mx = jnp.where(c4, mx_ew,
                   jnp.where(c3, mx_hi,
                             jnp.where(c2, mx_ns,
                                       jnp.where(c1, mx_lo, mx_ew))))
    thin = jnp.where(mq > mx, mag, 0.0)

    te = (jnp.where(thin > 0.5, 0.5, 0.0) + jnp.where(thin > 1.0, 0.5, 0.0))
    cs = _shl(te) + te + _shr(te)
    hs = _sup(cs) + cs + _sdn(cs)
    te_c = te[8:bh + 8]
    out_thin = jnp.where(
        te_c == 1.0, 1.0,
        jnp.where(te_c == 0.5,
                  jnp.where(hs[8:bh + 8] * 1.25 > 1.0, 1.0, 0.0), 0.0))

    ogx_ref[0, 0] = gx[8:bh + 8]
    ogy_ref[0, 0] = gy[8:bh + 8]
    omag_ref[0, 0] = mag[8:bh + 8]
    oori_ref[0, 0] = ori[8:bh + 8]
    ote_ref[0, 0] = out_thin


def _run(img):
    b, ch, h, w = img.shape
    bh = 256
    n_i = h // bh
    hb = bh // 8

    out_sds = jax.ShapeDtypeStruct((b, 1, h, w), jnp.float32)
    out_spec = pl.BlockSpec((1, 1, bh, w), lambda bi, i: (bi, 0, i, 0))
    grid = (b, n_i)
    fn = functools.partial(_canny_block, bh=bh, h=h, w=w)
    outs = pl.pallas_call(
        fn,
        grid=grid,
        in_specs=[
            pl.BlockSpec((1, ch, bh, w), lambda bi, i: (bi, 0, i, 0)),
            pl.BlockSpec((1, ch, 8, w),
                         lambda bi, i: (bi, 0, jnp.maximum(hb * i - 1, 0), 0)),
            pl.BlockSpec((1, ch, 8, w),
                         lambda bi, i: (bi, 0,
                                        jnp.minimum(hb * (i + 1), h // 8 - 1),
                                        0)),
        ],
        out_specs=[out_spec] * 5,
        out_shape=[out_sds] * 5,
        compiler_params=pltpu.CompilerParams(
            dimension_semantics=("parallel", "arbitrary")),
    )(img, img, img)
    return tuple(outs)


@jax.jit
def kernel(img):
    return _run(img)


# speedup vs baseline: 1.1442x; 1.0005x over previous
import functools

import jax
import jax.numpy as jnp
import numpy as np
from jax.experimental import pallas as pl
from jax.experimental.pallas import tpu as pltpu

_T = [float(np.tan((2 * j + 1) * np.pi / 16)) for j in range(4)]


def _shl(a):
    return jnp.concatenate([a[:, 1:], jnp.zeros_like(a[:, :1])], axis=1)


def _shr(a):
    return jnp.concatenate([jnp.zeros_like(a[:, :1]), a[:, :-1]], axis=1)


def _sup(a):
    return jnp.concatenate([a[1:], jnp.zeros_like(a[:1])], axis=0)


def _sdn(a):
    return jnp.concatenate([jnp.zeros_like(a[:1]), a[:-1]], axis=0)


def _canny_block(x_ref, t_ref, b_ref, ogx_ref, ogy_ref, omag_ref, oori_ref,
                 ote_ref, *, bh, h, w):
    i = pl.program_id(1)
    n_i = pl.num_programs(1)
    third = jnp.float32(1.0 / 3.0)
    wtop = jnp.where(i > 0, third, 0.0)
    wbot = jnp.where(i < n_i - 1, third, 0.0)

    def q(a):
        return a.astype(jnp.bfloat16).astype(jnp.float32)

    m_main = (q(x_ref[0, 0]) + q(x_ref[0, 1]) + q(x_ref[0, 2])) * third
    m_top = (q(t_ref[0, 0]) + q(t_ref[0, 1]) + q(t_ref[0, 2])) * wtop
    m_bot = (q(b_ref[0, 0]) + q(b_ref[0, 1]) + q(b_ref[0, 2])) * wbot
    m = jnp.concatenate([m_top, m_main, m_bot], axis=0)

    mu, md = _sup(m), _sdn(m)
    vsm = (mu + md) * 0.5 + m
    drow = mu - md
    gx = _shl(vsm) - _shr(vsm)
    gy = (_shl(drow) + _shr(drow)) * 0.5 + drow
    mag = jnp.sqrt(gx * gx + gy * gy)

    son = jnp.where(i > 0, 1.0, 0.0)
    sbn = jnp.where(i < n_i - 1, 1.0, 0.0)
    mag = jnp.concatenate(
        [mag[:8] * son, mag[8:bh + 8], mag[bh + 8:] * sbn], axis=0)

    t = gy / gx
    at = jnp.abs(t)
    tneg = t < 0
    c1, c2, c3, c4 = (at > _T[0]), (at > _T[1]), (at > _T[2]), (at > _T[3])
    u = (jnp.where(c1, 1.0, 0.0) + jnp.where(c2, 1.0, 0.0)
         + jnp.where(c3, 1.0, 0.0) + jnp.where(c4, 1.0, 0.0))
    kf = jnp.where(tneg, -u, u)
    ori = kf * 45.0 + 180.0

    mq = q(mag)
    mq_up, mq_dn = _sup(mq), _sdn(mq)
    n_e, n_w = _shl(mq), _shr(mq)
    n_ne, n_nw = _shl(mq_dn), _shr(mq_dn)
    n_se, n_sw = _shl(mq_up), _shr(mq_up)
    mx_ew = jnp.maximum(n_e, n_w)
    mx_ns = jnp.maximum(mq_dn, mq_up)
    mx_d1 = jnp.maximum(n_ne, n_sw)
    mx_d2 = jnp.maximum(n_nw, n_se)
    mx_lo = jnp.where(tneg, mx_d2, mx_d1)
    mx_hi = jnp.where(tneg, mx_d1, mx_d2)
    mx = jnp.where(c4, mx_ew,
                   jnp.where(c3, mx_hi,
                             jnp.where(c2, mx_ns,
                                       jnp.where(c1, mx_lo, mx_ew))))
    thin = jnp.where(mq > mx, mag, 0.0)

    te = (jnp.where(thin > 0.5, 0.5, 0.0) + jnp.where(thin > 1.0, 0.5, 0.0))
    cs = _shl(te) + te + _shr(te)
    hs = _sup(cs) + cs + _sdn(cs)
    te_c = te[8:bh + 8]
    out_thin = jnp.where(
        te_c == 1.0, 1.0,
        jnp.where(te_c == 0.5,
                  jnp.where(hs[8:bh + 8] * 1.25 > 1.0, 1.0, 0.0), 0.0))

    ogx_ref[0, 0] = gx[8:bh + 8]
    ogy_ref[0, 0] = gy[8:bh + 8]
    omag_ref[0, 0] = mag[8:bh + 8]
    oori_ref[0, 0] = ori[8:bh + 8]
    ote_ref[0, 0] = out_thin


def _run(img):
    b, ch, h, w = img.shape
    bh = 256
    n_i = h // bh
    hb = bh // 8

    out_sds = jax.ShapeDtypeStruct((b, 1, h, w), jnp.float32)
    out_spec = pl.BlockSpec((1, 1, bh, w), lambda bi, i: (bi, 0, i, 0))
    grid = (b, n_i)
    fn = functools.partial(_canny_block, bh=bh, h=h, w=w)
    outs = pl.pallas_call(
        fn,
        grid=grid,
        in_specs=[
            pl.BlockSpec((1, ch, bh, w), lambda bi, i: (bi, 0, i, 0)),
            pl.BlockSpec((1, ch, 8, w),
                         lambda bi, i: (bi, 0, jnp.maximum(hb * i - 1, 0), 0)),
            pl.BlockSpec((1, ch, 8, w),
                         lambda bi, i: (bi, 0,
                                        jnp.minimum(hb * (i + 1), h // 8 - 1),
                                        0)),
        ],
        out_specs=[out_spec] * 5,
        out_shape=[out_sds] * 5,
        compiler_params=pltpu.CompilerParams(
            dimension_semantics=("parallel", "arbitrary")),
    )(img, img, img)
    return tuple(outs)


@jax.jit
def kernel(img):
    return _run(img)
